# baseline (device time: 12454 ns/iter reference)
import jax
import jax.numpy as jnp
from jax import lax
from jax.experimental import pallas as pl
from jax.experimental.pallas import tpu as pltpu

N_GLOBAL = 1024
EPS = 1e-5


def kernel(x, gamma, beta):
    m, n = x.shape
    gamma2 = gamma.reshape(1, n)
    beta2 = beta.reshape(1, n)

    def body(x_ref, g_ref, b_ref, out_ref, send_buf, recv_buf, send_sem, recv_sem):
        my_x = lax.axis_index("x")
        my_y = lax.axis_index("y")
        peer = (my_x, 1 - my_y)

        barrier = pltpu.get_barrier_semaphore()
        pl.semaphore_signal(
            barrier, inc=1, device_id=peer, device_id_type=pl.DeviceIdType.MESH
        )
        pl.semaphore_wait(barrier, 1)

        xv = x_ref[:, :]
        s = jnp.sum(xv, axis=1, keepdims=True)
        sq = jnp.sum(xv * xv, axis=1, keepdims=True)
        send_buf[:, 0:1] = s
        send_buf[:, 1:2] = sq

        rdma = pltpu.make_async_remote_copy(
            src_ref=send_buf,
            dst_ref=recv_buf,
            send_sem=send_sem,
            recv_sem=recv_sem,
            device_id=peer,
            device_id_type=pl.DeviceIdType.MESH,
        )
        rdma.start()
        rdma.wait()

        tot_s = s + recv_buf[:, 0:1]
        tot_sq = sq + recv_buf[:, 1:2]
        mean = tot_s / N_GLOBAL
        var = tot_sq / N_GLOBAL - mean * mean
        inv = lax.rsqrt(var + EPS)
        out_ref[:, :] = g_ref[:, :] * (xv - mean) * inv + b_ref[:, :]

    return pl.pallas_call(
        body,
        out_shape=jax.ShapeDtypeStruct((m, n), x.dtype),
        in_specs=[
            pl.BlockSpec(memory_space=pltpu.VMEM),
            pl.BlockSpec(memory_space=pltpu.VMEM),
            pl.BlockSpec(memory_space=pltpu.VMEM),
        ],
        out_specs=pl.BlockSpec(memory_space=pltpu.VMEM),
        scratch_shapes=[
            pltpu.VMEM((m, 2), jnp.float32),
            pltpu.VMEM((m, 2), jnp.float32),
            pltpu.SemaphoreType.DMA,
            pltpu.SemaphoreType.DMA,
        ],
        compiler_params=pltpu.CompilerParams(collective_id=0),
    )(x, gamma2, beta2)
